# speedup vs baseline: 7.1866x; 7.1866x over previous
import jax
import jax.numpy as jnp
from jax.experimental import pallas as pl

D = 128
N_MAPPED = 50000
N_CAND = 100000
GRID = 50
BM = N_MAPPED // GRID
BU = N_CAND // GRID


def _body(xm_ref, xu_ref, seg_ref, bm_ref, bc_ref, bu_ref, out_ref):
    i = pl.program_id(0)

    @pl.when(i == 0)
    def _():
        out_ref[...] = jnp.concatenate(
            [bm_ref[...], bc_ref[...], bu_ref[...]], axis=0)

    xm = xm_ref[...]
    xu = xu_ref[...]
    segf = seg_ref[0].astype(jnp.float32)
    sm = jnp.sum(xm, axis=0, keepdims=True)
    st = jnp.sum(xu, axis=0, keepdims=True)
    s1 = jnp.dot(segf, xu, preferred_element_type=jnp.float32)
    out_ref[0:1, :] += sm
    out_ref[1:2, :] += st - s1
    out_ref[2:3, :] += s1


def kernel(X_mapped, X_unmapped, segment_ids, X_map_bias, X_connected_bias,
           X_unconnected_bias):
    seg3d = segment_ids.astype(jnp.int32).reshape(GRID, 1, BU)
    out = pl.pallas_call(
        _body,
        grid=(GRID,),
        in_specs=[
            pl.BlockSpec((BM, D), lambda i: (i, 0)),
            pl.BlockSpec((BU, D), lambda i: (i, 0)),
            pl.BlockSpec((1, 1, BU), lambda i: (i, 0, 0)),
            pl.BlockSpec((1, D), lambda i: (0, 0)),
            pl.BlockSpec((1, D), lambda i: (0, 0)),
            pl.BlockSpec((1, D), lambda i: (0, 0)),
        ],
        out_specs=pl.BlockSpec((3, D), lambda i: (0, 0)),
        out_shape=jax.ShapeDtypeStruct((3, D), jnp.float32),
    )(X_mapped, X_unmapped, seg3d, X_map_bias, X_connected_bias,
      X_unconnected_bias)
    return out.reshape(-1)
